# baseline (speedup 1.0000x reference)
import functools
import jax
import jax.numpy as jnp
import numpy as np
from jax import lax
from jax.experimental import pallas as pl
from jax.experimental.pallas import tpu as pltpu

IN_C = 32
OUT_C = 64
STRIDE = 4
RADIUS = 0.1
NSAMPLE = 32


def _fps(p, m):
    B, N, _ = p.shape
    dist0 = jnp.full((B, N), 1e10, jnp.float32)
    far0 = jnp.zeros((B,), jnp.int32)
    idx0 = jnp.zeros((B, m), jnp.int32)

    def body(i, st):
        dist, far, idx = st
        idx = idx.at[:, i].set(far)
        centroid = jnp.take_along_axis(p, far[:, None, None], axis=1)
        d = jnp.sum((p - centroid) ** 2, -1)
        dist = jnp.minimum(dist, d)
        far = jnp.argmax(dist, -1).astype(jnp.int32)
        return dist, far, idx

    _, _, idx = jax.lax.fori_loop(0, m, body, (dist0, far0, idx0))
    return idx


def _ball_query(p, new_p, radius, K):
    N = p.shape[1]
    sqr = jnp.sum((new_p[:, :, None, :] - p[:, None, :, :]) ** 2, -1)
    ar = jnp.arange(N, dtype=jnp.int32)
    cand = jnp.where(sqr <= radius * radius, ar[None, None, :], N)
    cand = jnp.sort(cand, axis=-1)[:, :, :K]
    first = cand[:, :, :1]
    cand = jnp.where(cand == N, first, cand)
    cand = jnp.where(cand == N, 0, cand)
    return cand.astype(jnp.int32)


def _conv_bn_act(x, W, gamma, beta):
    y = jnp.einsum('oc,bcmk->bomk', W, x)
    mean = jnp.mean(y, axis=(0, 2, 3), keepdims=True)
    var = jnp.var(y, axis=(0, 2, 3), keepdims=True)
    y = (y - mean) / jnp.sqrt(var + 1e-5)
    y = y * gamma[None, :, None, None] + beta[None, :, None, None]
    return jax.nn.relu(y)


def _maxpool_body(x_ref, o_ref):
    o_ref[...] = jnp.max(x_ref[...], axis=1)[:, None]


def _maxpool_k(x):
    R, K = x.shape
    blk = 1024
    out = pl.pallas_call(
        _maxpool_body,
        grid=(R // blk,),
        in_specs=[pl.BlockSpec((blk, K), lambda i: (i, 0))],
        out_specs=pl.BlockSpec((blk, 1), lambda i: (i, 0)),
        out_shape=jax.ShapeDtypeStruct((R, 1), x.dtype),
    )(x)
    return out[:, 0]


def kernel(p, f, W1, g1, b1, W2, g2, b2):
    B, N, _ = p.shape
    M = N // STRIDE
    idx = _fps(p, M)
    new_p = jnp.take_along_axis(p, idx[:, :, None], axis=1)
    gidx = _ball_query(p, new_p, RADIUS, NSAMPLE)
    bidx = jnp.arange(B)[:, None, None]
    grouped_p = p[bidx, gidx]
    dp = (grouped_p - new_p[:, :, None, :]).transpose(0, 3, 1, 2)
    fT = f.transpose(0, 2, 1)
    fj = fT[bidx, gidx].transpose(0, 3, 1, 2)
    x = jnp.concatenate([dp, fj], axis=1)
    x = _conv_bn_act(x, W1, g1, b1)
    x = _conv_bn_act(x, W2, g2, b2)
    f_out = _maxpool_k(x.reshape(B * OUT_C * M, NSAMPLE)).reshape(B, OUT_C, M)
    return (new_p, f_out)


# speedup vs baseline: 1.6498x; 1.6498x over previous
import functools
import jax
import jax.numpy as jnp
import numpy as np
from jax import lax
from jax.experimental import pallas as pl
from jax.experimental.pallas import tpu as pltpu

IN_C = 32
OUT_C = 64
STRIDE = 4
RADIUS = 0.1
NSAMPLE = 32


def _fps_body(px_ref, py_ref, pz_ref, idx_ref, npx_ref, npy_ref, npz_ref,
              *, M, N, SUB, MSUB):
    LN = N // SUB
    ML = M // MSUB
    px = px_ref[0]
    py = py_ref[0]
    pz = pz_ref[0]
    nidx = (lax.broadcasted_iota(jnp.int32, (SUB, LN), 0) * LN
            + lax.broadcasted_iota(jnp.int32, (SUB, LN), 1))
    midx = (lax.broadcasted_iota(jnp.int32, (MSUB, ML), 0) * ML
            + lax.broadcasted_iota(jnp.int32, (MSUB, ML), 1))

    def body(m, st):
        dist, far, aidx, ax, ay, az = st
        sel = midx == m
        aidx = jnp.where(sel, far, aidx)
        onehot = nidx == far
        cx = jnp.sum(jnp.where(onehot, px, 0.0))
        cy = jnp.sum(jnp.where(onehot, py, 0.0))
        cz = jnp.sum(jnp.where(onehot, pz, 0.0))
        ax = jnp.where(sel, cx, ax)
        ay = jnp.where(sel, cy, ay)
        az = jnp.where(sel, cz, az)
        dxv = px - cx
        dyv = py - cy
        dzv = pz - cz
        d = (dxv * dxv + dzv * dzv) + dyv * dyv
        dist = jnp.minimum(dist, d)
        mx = jnp.max(dist)
        far = jnp.min(jnp.where(dist == mx, nidx, N))
        return dist, far, aidx, ax, ay, az

    dist0 = jnp.full((SUB, LN), 1e10, jnp.float32)
    z = jnp.zeros((MSUB, ML), jnp.float32)
    zi = jnp.zeros((MSUB, ML), jnp.int32)
    _, _, aidx, ax, ay, az = lax.fori_loop(
        0, M, body, (dist0, jnp.int32(0), zi, z, z, z))
    idx_ref[0] = aidx
    npx_ref[0] = ax
    npy_ref[0] = ay
    npz_ref[0] = az


def _fps(p, M):
    B, N, _ = p.shape
    SUB = 8
    pr = p.transpose(0, 2, 1).reshape(B, 3, SUB, N // SUB)
    px, py, pz = pr[:, 0], pr[:, 1], pr[:, 2]
    MSUB = 16
    spec_in = pl.BlockSpec((1, SUB, N // SUB), lambda b: (b, 0, 0))
    spec_out = pl.BlockSpec((1, MSUB, M // MSUB), lambda b: (b, 0, 0))
    idx, npx, npy, npz = pl.pallas_call(
        functools.partial(_fps_body, M=M, N=N, SUB=SUB, MSUB=MSUB),
        grid=(B,),
        in_specs=[spec_in, spec_in, spec_in],
        out_specs=[spec_out, spec_out, spec_out, spec_out],
        out_shape=[
            jax.ShapeDtypeStruct((B, MSUB, M // MSUB), jnp.int32),
            jax.ShapeDtypeStruct((B, MSUB, M // MSUB), jnp.float32),
            jax.ShapeDtypeStruct((B, MSUB, M // MSUB), jnp.float32),
            jax.ShapeDtypeStruct((B, MSUB, M // MSUB), jnp.float32),
        ],
    )(px, py, pz)
    new_p = jnp.stack([npx.reshape(B, M), npy.reshape(B, M),
                       npz.reshape(B, M)], axis=-1)
    return idx.reshape(B, M), new_p


def _ball_query(p, new_p, radius, K):
    N = p.shape[1]
    sqr = jnp.sum((new_p[:, :, None, :] - p[:, None, :, :]) ** 2, -1)
    ar = jnp.arange(N, dtype=jnp.int32)
    cand = jnp.where(sqr <= radius * radius, ar[None, None, :], N)
    cand = jnp.sort(cand, axis=-1)[:, :, :K]
    first = cand[:, :, :1]
    cand = jnp.where(cand == N, first, cand)
    cand = jnp.where(cand == N, 0, cand)
    return cand.astype(jnp.int32)


def _conv_bn_act(x, W, gamma, beta):
    y = jnp.einsum('oc,bcmk->bomk', W, x)
    mean = jnp.mean(y, axis=(0, 2, 3), keepdims=True)
    var = jnp.var(y, axis=(0, 2, 3), keepdims=True)
    y = (y - mean) / jnp.sqrt(var + 1e-5)
    y = y * gamma[None, :, None, None] + beta[None, :, None, None]
    return jax.nn.relu(y)


def _maxpool_body(x_ref, o_ref):
    o_ref[...] = jnp.max(x_ref[...], axis=1)[:, None]


def _maxpool_k(x):
    R, K = x.shape
    blk = 1024
    out = pl.pallas_call(
        _maxpool_body,
        grid=(R // blk,),
        in_specs=[pl.BlockSpec((blk, K), lambda i: (i, 0))],
        out_specs=pl.BlockSpec((blk, 1), lambda i: (i, 0)),
        out_shape=jax.ShapeDtypeStruct((R, 1), x.dtype),
    )(x)
    return out[:, 0]


def kernel(p, f, W1, g1, b1, W2, g2, b2):
    B, N, _ = p.shape
    M = N // STRIDE
    idx, new_p = _fps(p, M)
    gidx = _ball_query(p, new_p, RADIUS, NSAMPLE)
    bidx = jnp.arange(B)[:, None, None]
    grouped_p = p[bidx, gidx]
    dp = (grouped_p - new_p[:, :, None, :]).transpose(0, 3, 1, 2)
    fT = f.transpose(0, 2, 1)
    fj = fT[bidx, gidx].transpose(0, 3, 1, 2)
    x = jnp.concatenate([dp, fj], axis=1)
    x = _conv_bn_act(x, W1, g1, b1)
    x = _conv_bn_act(x, W2, g2, b2)
    f_out = _maxpool_k(x.reshape(B * OUT_C * M, NSAMPLE)).reshape(B, OUT_C, M)
    return (new_p, f_out)


# speedup vs baseline: 3.1556x; 1.9127x over previous
import functools
import jax
import jax.numpy as jnp
import numpy as np
from jax import lax
from jax.experimental import pallas as pl
from jax.experimental.pallas import tpu as pltpu
from jax.experimental.pallas import tpu_sc as plsc

IN_C = 32
OUT_C = 64
STRIDE = 4
RADIUS = 0.1
NSAMPLE = 32


def _fps_body(px_ref, py_ref, pz_ref, idx_ref, npx_ref, npy_ref, npz_ref,
              *, M, N, SUB, MSUB):
    LN = N // SUB
    ML = M // MSUB
    px = px_ref[0]
    py = py_ref[0]
    pz = pz_ref[0]
    nidx = (lax.broadcasted_iota(jnp.int32, (SUB, LN), 0) * LN
            + lax.broadcasted_iota(jnp.int32, (SUB, LN), 1))
    midx = (lax.broadcasted_iota(jnp.int32, (MSUB, ML), 0) * ML
            + lax.broadcasted_iota(jnp.int32, (MSUB, ML), 1))

    def body(m, st):
        dist, far, aidx, ax, ay, az = st
        sel = midx == m
        aidx = jnp.where(sel, far, aidx)
        onehot = nidx == far
        cx = jnp.sum(jnp.where(onehot, px, 0.0))
        cy = jnp.sum(jnp.where(onehot, py, 0.0))
        cz = jnp.sum(jnp.where(onehot, pz, 0.0))
        ax = jnp.where(sel, cx, ax)
        ay = jnp.where(sel, cy, ay)
        az = jnp.where(sel, cz, az)
        dxv = px - cx
        dyv = py - cy
        dzv = pz - cz
        d = (dxv * dxv + dzv * dzv) + dyv * dyv
        dist = jnp.minimum(dist, d)
        mx = jnp.max(dist)
        far = jnp.min(jnp.where(dist == mx, nidx, N))
        return dist, far, aidx, ax, ay, az

    dist0 = jnp.full((SUB, LN), 1e10, jnp.float32)
    z = jnp.zeros((MSUB, ML), jnp.float32)
    zi = jnp.zeros((MSUB, ML), jnp.int32)
    _, _, aidx, ax, ay, az = lax.fori_loop(
        0, M, body, (dist0, jnp.int32(0), zi, z, z, z))
    idx_ref[0] = aidx
    npx_ref[0] = ax
    npy_ref[0] = ay
    npz_ref[0] = az


def _fps(p, M):
    B, N, _ = p.shape
    SUB = 8
    pr = p.transpose(0, 2, 1).reshape(B, 3, SUB, N // SUB)
    px, py, pz = pr[:, 0], pr[:, 1], pr[:, 2]
    MSUB = 16
    spec_in = pl.BlockSpec((1, SUB, N // SUB), lambda b: (b, 0, 0))
    spec_out = pl.BlockSpec((1, MSUB, M // MSUB), lambda b: (b, 0, 0))
    idx, npx, npy, npz = pl.pallas_call(
        functools.partial(_fps_body, M=M, N=N, SUB=SUB, MSUB=MSUB),
        grid=(B,),
        in_specs=[spec_in, spec_in, spec_in],
        out_specs=[spec_out, spec_out, spec_out, spec_out],
        out_shape=[
            jax.ShapeDtypeStruct((B, MSUB, M // MSUB), jnp.int32),
            jax.ShapeDtypeStruct((B, MSUB, M // MSUB), jnp.float32),
            jax.ShapeDtypeStruct((B, MSUB, M // MSUB), jnp.float32),
            jax.ShapeDtypeStruct((B, MSUB, M // MSUB), jnp.float32),
        ],
    )(px, py, pz)
    new_p = jnp.stack([npx.reshape(B, M), npy.reshape(B, M),
                       npz.reshape(B, M)], axis=-1)
    return idx.reshape(B, M), new_p


def _ball_query(p, new_p, radius, K):
    B, N, _ = p.shape
    M = new_p.shape[1]
    NW = 32
    CPT = (B * M) // NW
    PW = NW // B
    CH = 128
    NCH = N // CH
    CBUF = 128
    r2 = np.float32(radius * radius)

    mesh = plsc.VectorSubcoreMesh(core_axis_name="c", subcore_axis_name="s")

    @functools.partial(
        pl.kernel,
        out_type=jax.ShapeDtypeStruct((B * M * K,), jnp.int32),
        mesh=mesh,
        scratch_types=[
            pltpu.VMEM((N,), jnp.float32),
            pltpu.VMEM((N,), jnp.float32),
            pltpu.VMEM((N,), jnp.float32),
            pltpu.VMEM((CPT + 16,), jnp.float32),
            pltpu.VMEM((CPT + 16,), jnp.float32),
            pltpu.VMEM((CPT + 16,), jnp.float32),
            pltpu.VMEM((CBUF,), jnp.int32),
            pltpu.VMEM((CPT * K,), jnp.int32),
        ],
    )
    def bq(px_h, py_h, pz_h, cx_h, cy_h, cz_h, out_h,
           pxv, pyv, pzv, cxv, cyv, czv, cand, gbuf):
        wid = lax.axis_index("s") * 2 + lax.axis_index("c")
        b = wid // PW
        part = wid % PW
        pltpu.sync_copy(px_h.at[pl.ds(b * N, N)], pxv)
        pltpu.sync_copy(py_h.at[pl.ds(b * N, N)], pyv)
        pltpu.sync_copy(pz_h.at[pl.ds(b * N, N)], pzv)
        cbase = b * M + part * CPT
        pltpu.sync_copy(cx_h.at[pl.ds(cbase, CPT)], cxv.at[pl.ds(0, CPT)])
        pltpu.sync_copy(cy_h.at[pl.ds(cbase, CPT)], cyv.at[pl.ds(0, CPT)])
        pltpu.sync_copy(cz_h.at[pl.ds(cbase, CPT)], czv.at[pl.ds(0, CPT)])

        lanes = lax.iota(jnp.int32, 16)
        zeros16 = jnp.zeros((16,), jnp.int32)

        def per_centroid(i, carry):
            ccx = cxv[pl.ds(i, 16)][0]
            ccy = cyv[pl.ds(i, 16)][0]
            ccz = czv[pl.ds(i, 16)][0]
            cand[pl.ds(0, 16)] = zeros16

            one16 = jnp.ones((16,), jnp.int32)
            zero16 = jnp.zeros((16,), jnp.int32)

            def hsum(v):
                s = v[0]
                for l in range(1, 16):
                    s = s + v[l]
                return s

            BIG = jnp.int32(1 << 30)
            big16 = jnp.zeros((16,), jnp.int32) + BIG

            def chunk(j, cnt):
                base = j * CH
                mi = []
                for r in range(CH // 16):
                    off = base + r * 16
                    dx = pxv[pl.ds(off, 16)] - ccx
                    dy = pyv[pl.ds(off, 16)] - ccy
                    dz = pzv[pl.ds(off, 16)] - ccz
                    d2 = (dx * dx + dz * dz) + dy * dy
                    mi.append(jnp.where(d2 <= r2, one16, zero16))
                tot = mi[0]
                for r in range(1, CH // 16):
                    tot = tot + mi[r]
                s1 = hsum(tot)

                def one_hit(cnt):
                    wv = mi[0] * (lanes + base)
                    for r in range(1, CH // 16):
                        wv = wv + mi[r] * (lanes + (base + r * 16))
                    gidx = hsum(wv)
                    app = jnp.where(lanes == 0, gidx, BIG)
                    cand[pl.ds(jnp.minimum(cnt, 64), 16)] = app
                    return cnt + 1

                def multi_hit(cnt):
                    for r in range(CH // 16):
                        v = big16
                        rank = jnp.int32(0)
                        for l in range(16):
                            ml = mi[r][l]
                            val = jnp.where(ml > 0,
                                            jnp.int32(0) + (base + r * 16 + l),
                                            BIG)
                            v = jnp.where(lanes == rank, val, v)
                            rank = rank + ml
                        cand[pl.ds(jnp.minimum(cnt, 64), 16)] = v
                        cnt = cnt + rank
                    return cnt

                def any_hit(cnt):
                    return lax.cond(s1 == 1, one_hit, multi_hit, cnt)

                return lax.cond(s1 > 0, any_hit, lambda c: c, cnt)

            cnt = lax.fori_loop(0, NCH, chunk, jnp.int32(0))

            first = jnp.zeros((16,), jnp.int32) + cand[pl.ds(0, 16)][0]
            row0 = jnp.where(lanes < cnt, cand[pl.ds(0, 16)], first)
            row1 = jnp.where(lanes + 16 < cnt, cand[pl.ds(16, 16)], first)
            gbuf[pl.ds(i * K, 16)] = row0
            gbuf[pl.ds(i * K + 16, 16)] = row1
            return carry

        lax.fori_loop(0, CPT, per_centroid, jnp.int32(0))
        pltpu.sync_copy(gbuf, out_h.at[pl.ds(wid * CPT * K, CPT * K)])

    pT = p.transpose(2, 0, 1).reshape(3, B * N)
    cT = new_p.transpose(2, 0, 1).reshape(3, B * M)
    out = bq(pT[0], pT[1], pT[2], cT[0], cT[1], cT[2])
    return out.reshape(B, M, K)


def _conv_bn_act(x, W, gamma, beta):
    y = jnp.einsum('oc,bcmk->bomk', W, x)
    mean = jnp.mean(y, axis=(0, 2, 3), keepdims=True)
    var = jnp.var(y, axis=(0, 2, 3), keepdims=True)
    y = (y - mean) / jnp.sqrt(var + 1e-5)
    y = y * gamma[None, :, None, None] + beta[None, :, None, None]
    return jax.nn.relu(y)


def _maxpool_body(x_ref, o_ref):
    o_ref[...] = jnp.max(x_ref[...], axis=1)[:, None]


def _maxpool_k(x):
    R, K = x.shape
    blk = 1024
    out = pl.pallas_call(
        _maxpool_body,
        grid=(R // blk,),
        in_specs=[pl.BlockSpec((blk, K), lambda i: (i, 0))],
        out_specs=pl.BlockSpec((blk, 1), lambda i: (i, 0)),
        out_shape=jax.ShapeDtypeStruct((R, 1), x.dtype),
    )(x)
    return out[:, 0]


def kernel(p, f, W1, g1, b1, W2, g2, b2):
    B, N, _ = p.shape
    M = N // STRIDE
    idx, new_p = _fps(p, M)
    del idx
    gidx = _ball_query(p, new_p, RADIUS, NSAMPLE)
    bidx = jnp.arange(B)[:, None, None]
    grouped_p = p[bidx, gidx]
    dp = (grouped_p - new_p[:, :, None, :]).transpose(0, 3, 1, 2)
    fT = f.transpose(0, 2, 1)
    fj = fT[bidx, gidx].transpose(0, 3, 1, 2)
    x = jnp.concatenate([dp, fj], axis=1)
    x = _conv_bn_act(x, W1, g1, b1)
    x = _conv_bn_act(x, W2, g2, b2)
    f_out = _maxpool_k(x.reshape(B * OUT_C * M, NSAMPLE)).reshape(B, OUT_C, M)
    return (new_p, f_out)


# speedup vs baseline: 8.3839x; 2.6568x over previous
import functools
import jax
import jax.numpy as jnp
import numpy as np
from jax import lax
from jax.experimental import pallas as pl
from jax.experimental.pallas import tpu as pltpu
from jax.experimental.pallas import tpu_sc as plsc

IN_C = 32
OUT_C = 64
STRIDE = 4
RADIUS = 0.1
NSAMPLE = 32


def _fps_body(px_ref, py_ref, pz_ref, idx_ref, npx_ref, npy_ref, npz_ref,
              *, M, N, SUB, MSUB):
    LN = N // SUB
    ML = M // MSUB
    px = px_ref[0]
    py = py_ref[0]
    pz = pz_ref[0]
    nidx = (lax.broadcasted_iota(jnp.int32, (SUB, LN), 0) * LN
            + lax.broadcasted_iota(jnp.int32, (SUB, LN), 1))
    midx = (lax.broadcasted_iota(jnp.int32, (MSUB, ML), 0) * ML
            + lax.broadcasted_iota(jnp.int32, (MSUB, ML), 1))

    def body(m, st):
        dist, far, aidx, ax, ay, az = st
        sel = midx == m
        aidx = jnp.where(sel, far, aidx)
        onehot = nidx == far
        cx = jnp.sum(jnp.where(onehot, px, 0.0))
        cy = jnp.sum(jnp.where(onehot, py, 0.0))
        cz = jnp.sum(jnp.where(onehot, pz, 0.0))
        ax = jnp.where(sel, cx, ax)
        ay = jnp.where(sel, cy, ay)
        az = jnp.where(sel, cz, az)
        dxv = px - cx
        dyv = py - cy
        dzv = pz - cz
        d = (dxv * dxv + dzv * dzv) + dyv * dyv
        dist = jnp.minimum(dist, d)
        mx = jnp.max(dist)
        far = jnp.min(jnp.where(dist == mx, nidx, N))
        return dist, far, aidx, ax, ay, az

    dist0 = jnp.full((SUB, LN), 1e10, jnp.float32)
    z = jnp.zeros((MSUB, ML), jnp.float32)
    zi = jnp.zeros((MSUB, ML), jnp.int32)
    _, _, aidx, ax, ay, az = lax.fori_loop(
        0, M, body, (dist0, jnp.int32(0), zi, z, z, z))
    idx_ref[0] = aidx
    npx_ref[0] = ax
    npy_ref[0] = ay
    npz_ref[0] = az


def _fps(p, M):
    B, N, _ = p.shape
    SUB = 8
    pr = p.transpose(0, 2, 1).reshape(B, 3, SUB, N // SUB)
    px, py, pz = pr[:, 0], pr[:, 1], pr[:, 2]
    MSUB = 16
    spec_in = pl.BlockSpec((1, SUB, N // SUB), lambda b: (b, 0, 0))
    spec_out = pl.BlockSpec((1, MSUB, M // MSUB), lambda b: (b, 0, 0))
    idx, npx, npy, npz = pl.pallas_call(
        functools.partial(_fps_body, M=M, N=N, SUB=SUB, MSUB=MSUB),
        grid=(B,),
        in_specs=[spec_in, spec_in, spec_in],
        out_specs=[spec_out, spec_out, spec_out, spec_out],
        out_shape=[
            jax.ShapeDtypeStruct((B, MSUB, M // MSUB), jnp.int32),
            jax.ShapeDtypeStruct((B, MSUB, M // MSUB), jnp.float32),
            jax.ShapeDtypeStruct((B, MSUB, M // MSUB), jnp.float32),
            jax.ShapeDtypeStruct((B, MSUB, M // MSUB), jnp.float32),
        ],
    )(px, py, pz)
    new_p = jnp.stack([npx.reshape(B, M), npy.reshape(B, M),
                       npz.reshape(B, M)], axis=-1)
    return idx.reshape(B, M), new_p


def _ball_query(p, new_p, radius, K):
    B, N, _ = p.shape
    M = new_p.shape[1]
    NW = 32
    CPT = (B * M) // NW
    PW = NW // B
    CH = 128
    NCH = N // CH
    CBUF = 128
    r2 = np.float32(radius * radius)

    mesh = plsc.VectorSubcoreMesh(core_axis_name="c", subcore_axis_name="s")

    @functools.partial(
        pl.kernel,
        out_type=jax.ShapeDtypeStruct((B * M * K,), jnp.int32),
        mesh=mesh,
        scratch_types=[
            pltpu.VMEM((N,), jnp.float32),
            pltpu.VMEM((N,), jnp.float32),
            pltpu.VMEM((N,), jnp.float32),
            pltpu.VMEM((CPT + 16,), jnp.float32),
            pltpu.VMEM((CPT + 16,), jnp.float32),
            pltpu.VMEM((CPT + 16,), jnp.float32),
            pltpu.VMEM((CBUF,), jnp.int32),
            pltpu.VMEM((CPT * K,), jnp.int32),
        ],
    )
    def bq(px_h, py_h, pz_h, cx_h, cy_h, cz_h, out_h,
           pxv, pyv, pzv, cxv, cyv, czv, cand, gbuf):
        wid = lax.axis_index("s") * 2 + lax.axis_index("c")
        b = wid // PW
        part = wid % PW
        pltpu.sync_copy(px_h.at[pl.ds(b * N, N)], pxv)
        pltpu.sync_copy(py_h.at[pl.ds(b * N, N)], pyv)
        pltpu.sync_copy(pz_h.at[pl.ds(b * N, N)], pzv)
        cbase = b * M + part * CPT
        pltpu.sync_copy(cx_h.at[pl.ds(cbase, CPT)], cxv.at[pl.ds(0, CPT)])
        pltpu.sync_copy(cy_h.at[pl.ds(cbase, CPT)], cyv.at[pl.ds(0, CPT)])
        pltpu.sync_copy(cz_h.at[pl.ds(cbase, CPT)], czv.at[pl.ds(0, CPT)])

        lanes = lax.iota(jnp.int32, 16)
        zeros16 = jnp.zeros((16,), jnp.int32)

        bofs = b * N

        def per_centroid(i, carry):
            ccx = cxv[pl.ds(i, 16)][0]
            ccy = cyv[pl.ds(i, 16)][0]
            ccz = czv[pl.ds(i, 16)][0]
            cand[pl.ds(0, 16)] = zeros16 + bofs

            one16 = jnp.ones((16,), jnp.int32)
            zero16 = jnp.zeros((16,), jnp.int32)

            def hsum(v):
                s = v[0]
                for l in range(1, 16):
                    s = s + v[l]
                return s

            BIG = jnp.int32(1 << 30)
            big16 = jnp.zeros((16,), jnp.int32) + BIG

            def chunk(j, cnt):
                base = j * CH
                mi = []
                for r in range(CH // 16):
                    off = base + r * 16
                    dx = pxv[pl.ds(off, 16)] - ccx
                    dy = pyv[pl.ds(off, 16)] - ccy
                    dz = pzv[pl.ds(off, 16)] - ccz
                    d2 = (dx * dx + dz * dz) + dy * dy
                    mi.append(jnp.where(d2 <= r2, one16, zero16))
                tot = mi[0]
                for r in range(1, CH // 16):
                    tot = tot + mi[r]
                s1 = hsum(tot)

                def one_hit(cnt):
                    wv = mi[0] * (lanes + base)
                    for r in range(1, CH // 16):
                        wv = wv + mi[r] * (lanes + (base + r * 16))
                    gidx = hsum(wv) + bofs
                    app = jnp.where(lanes == 0, gidx, BIG)
                    cand[pl.ds(jnp.minimum(cnt, 64), 16)] = app
                    return cnt + 1

                def multi_hit(cnt):
                    for r in range(CH // 16):
                        v = big16
                        rank = jnp.int32(0)
                        for l in range(16):
                            ml = mi[r][l]
                            val = jnp.where(ml > 0,
                                            bofs + (base + r * 16 + l),
                                            BIG)
                            v = jnp.where(lanes == rank, val, v)
                            rank = rank + ml
                        cand[pl.ds(jnp.minimum(cnt, 64), 16)] = v
                        cnt = cnt + rank
                    return cnt

                def any_hit(cnt):
                    return lax.cond(s1 == 1, one_hit, multi_hit, cnt)

                return lax.cond(s1 > 0, any_hit, lambda c: c, cnt)

            cnt = lax.fori_loop(0, NCH, chunk, jnp.int32(0))

            first = jnp.zeros((16,), jnp.int32) + cand[pl.ds(0, 16)][0]
            row0 = jnp.where(lanes < cnt, cand[pl.ds(0, 16)], first)
            row1 = jnp.where(lanes + 16 < cnt, cand[pl.ds(16, 16)], first)
            gbuf[pl.ds(i * K, 16)] = row0
            gbuf[pl.ds(i * K + 16, 16)] = row1
            return carry

        lax.fori_loop(0, CPT, per_centroid, jnp.int32(0))
        pltpu.sync_copy(gbuf, out_h.at[pl.ds(wid * CPT * K, CPT * K)])

    pT = p.transpose(2, 0, 1).reshape(3, B * N)
    cT = new_p.transpose(2, 0, 1).reshape(3, B * M)
    return bq(pT[0], pT[1], pT[2], cT[0], cT[1], cT[2])


def _group_gather(T, gidx_flat, new_p):
    BN, D = T.shape
    B, M, _ = new_p.shape
    R = gidx_flat.shape[0]
    K = R // (B * M)
    NW = 32
    CPT = (B * M) // NW
    RPT = CPT * K
    SUB = 32
    SROWS = SUB * K

    mesh = plsc.VectorSubcoreMesh(core_axis_name="c", subcore_axis_name="s")

    @functools.partial(
        pl.kernel,
        out_type=jax.ShapeDtypeStruct((R, D), jnp.float32),
        mesh=mesh,
        scratch_types=[
            pltpu.VMEM((RPT,), jnp.int32),
            pltpu.VMEM((CPT + 16,), jnp.float32),
            pltpu.VMEM((CPT + 16,), jnp.float32),
            pltpu.VMEM((CPT + 16,), jnp.float32),
            pltpu.VMEM((SROWS, D), jnp.float32),
            pltpu.SemaphoreType.DMA,
        ],
        compiler_params=pltpu.CompilerParams(use_tc_tiling_on_sc=False),
    )
    def gg(t_h, gi_h, cx_h, cy_h, cz_h, x_h, giv, cxv, cyv, czv, xbuf, sem):
        wid = lax.axis_index("s") * 2 + lax.axis_index("c")
        pltpu.sync_copy(gi_h.at[pl.ds(wid * RPT, RPT)], giv)
        cb = wid * CPT
        pltpu.sync_copy(cx_h.at[pl.ds(cb, CPT)], cxv.at[pl.ds(0, CPT)])
        pltpu.sync_copy(cy_h.at[pl.ds(cb, CPT)], cyv.at[pl.ds(0, CPT)])
        pltpu.sync_copy(cz_h.at[pl.ds(cb, CPT)], czv.at[pl.ds(0, CPT)])

        lanes = lax.iota(jnp.int32, 16)
        zf = jnp.zeros((16,), jnp.float32)

        def sub_chunk(s, carry):
            cps = []
            for q in range(SROWS // 128):
                idxs = giv.at[pl.ds(s * SROWS + q * 128, 128)]
                cps.append(pltpu.async_copy(
                    t_h.at[idxs], xbuf.at[pl.ds(q * 128, 128)], sem))
            for cp in cps:
                cp.wait()
            def fix_c(ci, carry2):
                c = s * SUB + ci
                ccx = cxv[pl.ds(c, 16)][0]
                ccy = cyv[pl.ds(c, 16)][0]
                ccz = czv[pl.ds(c, 16)][0]
                cvec = jnp.where(lanes == 0, ccx,
                                 jnp.where(lanes == 1, ccy,
                                           jnp.where(lanes == 2, ccz, zf)))
                for r in range(K):
                    row = ci * K + r
                    v = xbuf[row, pl.ds(0, 16)]
                    xbuf[row, pl.ds(0, 16)] = v - cvec
                return carry2

            lax.fori_loop(0, SUB, fix_c, jnp.int32(0))
            pltpu.sync_copy(
                xbuf, x_h.at[pl.ds(wid * RPT + s * SROWS, SROWS)])
            return carry

        lax.fori_loop(0, CPT // SUB, sub_chunk, jnp.int32(0))

    cT = new_p.transpose(2, 0, 1).reshape(3, B * M)
    return gg(T, gidx_flat, cT[0], cT[1], cT[2])


def _conv_bn_act(x, W, gamma, beta):
    y = jnp.einsum('oc,bcmk->bomk', W, x)
    mean = jnp.mean(y, axis=(0, 2, 3), keepdims=True)
    var = jnp.var(y, axis=(0, 2, 3), keepdims=True)
    y = (y - mean) / jnp.sqrt(var + 1e-5)
    y = y * gamma[None, :, None, None] + beta[None, :, None, None]
    return jax.nn.relu(y)


def _maxpool_body(x_ref, o_ref):
    o_ref[...] = jnp.max(x_ref[...], axis=1)[:, None]


def _maxpool_k(x):
    R, K = x.shape
    blk = 1024
    out = pl.pallas_call(
        _maxpool_body,
        grid=(R // blk,),
        in_specs=[pl.BlockSpec((blk, K), lambda i: (i, 0))],
        out_specs=pl.BlockSpec((blk, 1), lambda i: (i, 0)),
        out_shape=jax.ShapeDtypeStruct((R, 1), x.dtype),
    )(x)
    return out[:, 0]


def kernel(p, f, W1, g1, b1, W2, g2, b2):
    B, N, _ = p.shape
    M = N // STRIDE
    idx, new_p = _fps(p, M)
    del idx
    gidx = _ball_query(p, new_p, RADIUS, NSAMPLE)
    D = 40
    T = jnp.concatenate(
        [p, f.transpose(0, 2, 1),
         jnp.zeros((B, N, D - 3 - IN_C), jnp.float32)],
        axis=-1).reshape(B * N, D)
    X = _group_gather(T, gidx, new_p)
    x2 = X.reshape(B, M, NSAMPLE, D)
    W1p = jnp.pad(W1, ((0, 0), (0, D - W1.shape[1])))
    y = jnp.einsum('oc,bmkc->bomk', W1p, x2)
    mean = jnp.mean(y, axis=(0, 2, 3), keepdims=True)
    var = jnp.var(y, axis=(0, 2, 3), keepdims=True)
    y = (y - mean) / jnp.sqrt(var + 1e-5)
    y = jax.nn.relu(y * g1[None, :, None, None] + b1[None, :, None, None])
    x = _conv_bn_act(y, W2, g2, b2)
    f_out = _maxpool_k(x.reshape(B * OUT_C * M, NSAMPLE)).reshape(B, OUT_C, M)
    return (new_p, f_out)


# speedup vs baseline: 9.6096x; 1.1462x over previous
import functools
import jax
import jax.numpy as jnp
import numpy as np
from jax import lax
from jax.experimental import pallas as pl
from jax.experimental.pallas import tpu as pltpu
from jax.experimental.pallas import tpu_sc as plsc

IN_C = 32
OUT_C = 64
STRIDE = 4
RADIUS = 0.1
NSAMPLE = 32


def _fps_body(px_ref, py_ref, pz_ref, idx_ref, npx_ref, npy_ref, npz_ref,
              *, M, N, SUB, MSUB):
    LN = N // SUB
    ML = M // MSUB
    px = px_ref[0]
    py = py_ref[0]
    pz = pz_ref[0]
    nidx = (lax.broadcasted_iota(jnp.int32, (SUB, LN), 0) * LN
            + lax.broadcasted_iota(jnp.int32, (SUB, LN), 1))
    midx = (lax.broadcasted_iota(jnp.int32, (MSUB, ML), 0) * ML
            + lax.broadcasted_iota(jnp.int32, (MSUB, ML), 1))

    def body(m, st):
        dist, far, aidx, ax, ay, az = st
        sel = midx == m
        aidx = jnp.where(sel, far, aidx)
        onehot = nidx == far
        cx = jnp.sum(jnp.where(onehot, px, 0.0))
        cy = jnp.sum(jnp.where(onehot, py, 0.0))
        cz = jnp.sum(jnp.where(onehot, pz, 0.0))
        ax = jnp.where(sel, cx, ax)
        ay = jnp.where(sel, cy, ay)
        az = jnp.where(sel, cz, az)
        dxv = px - cx
        dyv = py - cy
        dzv = pz - cz
        d = (dxv * dxv + dzv * dzv) + dyv * dyv
        dist = jnp.minimum(dist, d)
        mx = jnp.max(dist)
        far = jnp.min(jnp.where(dist == mx, nidx, N))
        return dist, far, aidx, ax, ay, az

    dist0 = jnp.full((SUB, LN), 1e10, jnp.float32)
    z = jnp.zeros((MSUB, ML), jnp.float32)
    zi = jnp.zeros((MSUB, ML), jnp.int32)
    _, _, aidx, ax, ay, az = lax.fori_loop(
        0, M, body, (dist0, jnp.int32(0), zi, z, z, z))
    idx_ref[0] = aidx
    npx_ref[0] = ax
    npy_ref[0] = ay
    npz_ref[0] = az


def _fps_body2(px_ref, py_ref, pz_ref, idx_ref, npx_ref, npy_ref, npz_ref,
               *, M, N, B, SUB, MSUB):
    LN = N // SUB
    ML = M // MSUB
    pxs = [px_ref[pl.ds(b * SUB, SUB), :] for b in range(B)]
    pys = [py_ref[pl.ds(b * SUB, SUB), :] for b in range(B)]
    pzs = [pz_ref[pl.ds(b * SUB, SUB), :] for b in range(B)]
    nidx = (lax.broadcasted_iota(jnp.int32, (SUB, LN), 0) * LN
            + lax.broadcasted_iota(jnp.int32, (SUB, LN), 1))
    midx = (lax.broadcasted_iota(jnp.int32, (MSUB, ML), 0) * ML
            + lax.broadcasted_iota(jnp.int32, (MSUB, ML), 1))

    def body(m, st):
        dists, fars, aidxs, axs, ays, azs = st
        sel = midx == m
        out = ([], [], [], [], [], [])
        for b in range(B):
            dist, far = dists[b], fars[b]
            px, py, pz = pxs[b], pys[b], pzs[b]
            aidx = jnp.where(sel, far, aidxs[b])
            onehot = nidx == far
            cx = jnp.sum(jnp.where(onehot, px, 0.0))
            cy = jnp.sum(jnp.where(onehot, py, 0.0))
            cz = jnp.sum(jnp.where(onehot, pz, 0.0))
            ax = jnp.where(sel, cx, axs[b])
            ay = jnp.where(sel, cy, ays[b])
            az = jnp.where(sel, cz, azs[b])
            dxv = px - cx
            dyv = py - cy
            dzv = pz - cz
            d = (dxv * dxv + dzv * dzv) + dyv * dyv
            dist = jnp.minimum(dist, d)
            mx = jnp.max(dist)
            far = jnp.min(jnp.where(dist == mx, nidx, N))
            for lst, v in zip(out, (dist, far, aidx, ax, ay, az)):
                lst.append(v)
        return out

    dist0 = [jnp.full((SUB, LN), 1e10, jnp.float32)] * B
    far0 = [jnp.int32(0)] * B
    z = [jnp.zeros((MSUB, ML), jnp.float32)] * B
    zi = [jnp.zeros((MSUB, ML), jnp.int32)] * B
    _, _, aidxs, axs, ays, azs = lax.fori_loop(
        0, M, body, (dist0, far0, zi, z, z, z))
    for b in range(B):
        idx_ref[b] = aidxs[b]
        npx_ref[b] = axs[b]
        npy_ref[b] = ays[b]
        npz_ref[b] = azs[b]


def _fps(p, M):
    B, N, _ = p.shape
    SUB = 8
    MSUB = 16
    pr = p.transpose(0, 2, 1).reshape(B, 3, SUB, N // SUB)
    px = pr[:, 0].reshape(B * SUB, N // SUB)
    py = pr[:, 1].reshape(B * SUB, N // SUB)
    pz = pr[:, 2].reshape(B * SUB, N // SUB)
    idx, npx, npy, npz = pl.pallas_call(
        functools.partial(_fps_body2, M=M, N=N, B=B, SUB=SUB, MSUB=MSUB),
        out_shape=[
            jax.ShapeDtypeStruct((B, MSUB, M // MSUB), jnp.int32),
            jax.ShapeDtypeStruct((B, MSUB, M // MSUB), jnp.float32),
            jax.ShapeDtypeStruct((B, MSUB, M // MSUB), jnp.float32),
            jax.ShapeDtypeStruct((B, MSUB, M // MSUB), jnp.float32),
        ],
    )(px, py, pz)
    new_p = jnp.stack([npx.reshape(B, M), npy.reshape(B, M),
                       npz.reshape(B, M)], axis=-1)
    return idx.reshape(B, M), new_p


def _ball_query(p, new_p, radius, K):
    B, N, _ = p.shape
    M = new_p.shape[1]
    NW = 32
    CPT = (B * M) // NW
    PW = NW // B
    CH = 128
    NCH = N // CH
    CBUF = 128
    r2 = np.float32(radius * radius)

    mesh = plsc.VectorSubcoreMesh(core_axis_name="c", subcore_axis_name="s")

    @functools.partial(
        pl.kernel,
        out_type=jax.ShapeDtypeStruct((B * M * K,), jnp.int32),
        mesh=mesh,
        scratch_types=[
            pltpu.VMEM((N,), jnp.float32),
            pltpu.VMEM((N,), jnp.float32),
            pltpu.VMEM((N,), jnp.float32),
            pltpu.VMEM((CPT + 16,), jnp.float32),
            pltpu.VMEM((CPT + 16,), jnp.float32),
            pltpu.VMEM((CPT + 16,), jnp.float32),
            pltpu.VMEM((CBUF,), jnp.int32),
            pltpu.VMEM((CPT * K,), jnp.int32),
        ],
    )
    def bq(px_h, py_h, pz_h, cx_h, cy_h, cz_h, out_h,
           pxv, pyv, pzv, cxv, cyv, czv, cand, gbuf):
        wid = lax.axis_index("s") * 2 + lax.axis_index("c")
        b = wid // PW
        part = wid % PW
        pltpu.sync_copy(px_h.at[pl.ds(b * N, N)], pxv)
        pltpu.sync_copy(py_h.at[pl.ds(b * N, N)], pyv)
        pltpu.sync_copy(pz_h.at[pl.ds(b * N, N)], pzv)
        cbase = b * M + part * CPT
        pltpu.sync_copy(cx_h.at[pl.ds(cbase, CPT)], cxv.at[pl.ds(0, CPT)])
        pltpu.sync_copy(cy_h.at[pl.ds(cbase, CPT)], cyv.at[pl.ds(0, CPT)])
        pltpu.sync_copy(cz_h.at[pl.ds(cbase, CPT)], czv.at[pl.ds(0, CPT)])

        lanes = lax.iota(jnp.int32, 16)
        zeros16 = jnp.zeros((16,), jnp.int32)

        bofs = b * N

        def per_centroid(i, carry):
            ccx = cxv[pl.ds(i, 16)][0]
            ccy = cyv[pl.ds(i, 16)][0]
            ccz = czv[pl.ds(i, 16)][0]
            cand[pl.ds(0, 16)] = zeros16 + bofs

            one16 = jnp.ones((16,), jnp.int32)
            zero16 = jnp.zeros((16,), jnp.int32)

            def hsum(v):
                s = v[0]
                for l in range(1, 16):
                    s = s + v[l]
                return s

            BIG = jnp.int32(1 << 30)
            big16 = jnp.zeros((16,), jnp.int32) + BIG

            def chunk(j, cnt):
                base = j * CH
                mi = []
                for r in range(CH // 16):
                    off = base + r * 16
                    dx = pxv[pl.ds(off, 16)] - ccx
                    dy = pyv[pl.ds(off, 16)] - ccy
                    dz = pzv[pl.ds(off, 16)] - ccz
                    d2 = (dx * dx + dz * dz) + dy * dy
                    mi.append(jnp.where(d2 <= r2, one16, zero16))
                tot = mi[0]
                for r in range(1, CH // 16):
                    tot = tot + mi[r]
                s1 = hsum(tot)

                def one_hit(cnt):
                    wv = mi[0] * (lanes + base)
                    for r in range(1, CH // 16):
                        wv = wv + mi[r] * (lanes + (base + r * 16))
                    gidx = hsum(wv) + bofs
                    app = jnp.where(lanes == 0, gidx, BIG)
                    cand[pl.ds(jnp.minimum(cnt, 64), 16)] = app
                    return cnt + 1

                def multi_hit(cnt):
                    for r in range(CH // 16):
                        v = big16
                        rank = jnp.int32(0)
                        for l in range(16):
                            ml = mi[r][l]
                            val = jnp.where(ml > 0,
                                            bofs + (base + r * 16 + l),
                                            BIG)
                            v = jnp.where(lanes == rank, val, v)
                            rank = rank + ml
                        cand[pl.ds(jnp.minimum(cnt, 64), 16)] = v
                        cnt = cnt + rank
                    return cnt

                def any_hit(cnt):
                    return lax.cond(s1 == 1, one_hit, multi_hit, cnt)

                return lax.cond(s1 > 0, any_hit, lambda c: c, cnt)

            cnt = lax.fori_loop(0, NCH, chunk, jnp.int32(0))

            first = jnp.zeros((16,), jnp.int32) + cand[pl.ds(0, 16)][0]
            row0 = jnp.where(lanes < cnt, cand[pl.ds(0, 16)], first)
            row1 = jnp.where(lanes + 16 < cnt, cand[pl.ds(16, 16)], first)
            gbuf[pl.ds(i * K, 16)] = row0
            gbuf[pl.ds(i * K + 16, 16)] = row1
            return carry

        lax.fori_loop(0, CPT, per_centroid, jnp.int32(0))
        pltpu.sync_copy(gbuf, out_h.at[pl.ds(wid * CPT * K, CPT * K)])

    pT = p.transpose(2, 0, 1).reshape(3, B * N)
    cT = new_p.transpose(2, 0, 1).reshape(3, B * M)
    return bq(pT[0], pT[1], pT[2], cT[0], cT[1], cT[2])


def _group_gather(T, gidx_flat, new_p):
    BN, D = T.shape
    B, M, _ = new_p.shape
    R = gidx_flat.shape[0]
    K = R // (B * M)
    NW = 32
    CPT = (B * M) // NW
    RPT = CPT * K
    SUB = 32
    SROWS = SUB * K

    mesh = plsc.VectorSubcoreMesh(core_axis_name="c", subcore_axis_name="s")

    @functools.partial(
        pl.kernel,
        out_type=jax.ShapeDtypeStruct((R, D), jnp.float32),
        mesh=mesh,
        scratch_types=[
            pltpu.VMEM((RPT,), jnp.int32),
            pltpu.VMEM((CPT + 16,), jnp.float32),
            pltpu.VMEM((CPT + 16,), jnp.float32),
            pltpu.VMEM((CPT + 16,), jnp.float32),
            pltpu.VMEM((SROWS, D), jnp.float32),
            pltpu.SemaphoreType.DMA,
        ],
        compiler_params=pltpu.CompilerParams(use_tc_tiling_on_sc=False),
    )
    def gg(t_h, gi_h, cx_h, cy_h, cz_h, x_h, giv, cxv, cyv, czv, xbuf, sem):
        wid = lax.axis_index("s") * 2 + lax.axis_index("c")
        pltpu.sync_copy(gi_h.at[pl.ds(wid * RPT, RPT)], giv)
        cb = wid * CPT
        pltpu.sync_copy(cx_h.at[pl.ds(cb, CPT)], cxv.at[pl.ds(0, CPT)])
        pltpu.sync_copy(cy_h.at[pl.ds(cb, CPT)], cyv.at[pl.ds(0, CPT)])
        pltpu.sync_copy(cz_h.at[pl.ds(cb, CPT)], czv.at[pl.ds(0, CPT)])

        lanes = lax.iota(jnp.int32, 16)
        zf = jnp.zeros((16,), jnp.float32)

        def sub_chunk(s, carry):
            cps = []
            for q in range(SROWS // 128):
                idxs = giv.at[pl.ds(s * SROWS + q * 128, 128)]
                cps.append(pltpu.async_copy(
                    t_h.at[idxs], xbuf.at[pl.ds(q * 128, 128)], sem))
            for cp in cps:
                cp.wait()
            def fix_c(ci, carry2):
                c = s * SUB + ci
                ccx = cxv[pl.ds(c, 16)][0]
                ccy = cyv[pl.ds(c, 16)][0]
                ccz = czv[pl.ds(c, 16)][0]
                cvec = jnp.where(lanes == 0, ccx,
                                 jnp.where(lanes == 1, ccy,
                                           jnp.where(lanes == 2, ccz, zf)))
                for r in range(K):
                    row = ci * K + r
                    v = xbuf[row, pl.ds(0, 16)]
                    xbuf[row, pl.ds(0, 16)] = v - cvec
                return carry2

            lax.fori_loop(0, SUB, fix_c, jnp.int32(0))
            pltpu.sync_copy(
                xbuf, x_h.at[pl.ds(wid * RPT + s * SROWS, SROWS)])
            return carry

        lax.fori_loop(0, CPT // SUB, sub_chunk, jnp.int32(0))

    cT = new_p.transpose(2, 0, 1).reshape(3, B * M)
    return gg(T, gidx_flat, cT[0], cT[1], cT[2])


def _conv_bn_act(x, W, gamma, beta):
    y = jnp.einsum('oc,bcmk->bomk', W, x)
    mean = jnp.mean(y, axis=(0, 2, 3), keepdims=True)
    var = jnp.var(y, axis=(0, 2, 3), keepdims=True)
    y = (y - mean) / jnp.sqrt(var + 1e-5)
    y = y * gamma[None, :, None, None] + beta[None, :, None, None]
    return jax.nn.relu(y)


def _maxpool_body(x_ref, o_ref):
    o_ref[...] = jnp.max(x_ref[...], axis=1)[:, None]


def _maxpool_k(x):
    R, K = x.shape
    blk = 1024
    out = pl.pallas_call(
        _maxpool_body,
        grid=(R // blk,),
        in_specs=[pl.BlockSpec((blk, K), lambda i: (i, 0))],
        out_specs=pl.BlockSpec((blk, 1), lambda i: (i, 0)),
        out_shape=jax.ShapeDtypeStruct((R, 1), x.dtype),
    )(x)
    return out[:, 0]


def kernel(p, f, W1, g1, b1, W2, g2, b2):
    B, N, _ = p.shape
    M = N // STRIDE
    idx, new_p = _fps(p, M)
    del idx
    gidx = _ball_query(p, new_p, RADIUS, NSAMPLE)
    D = 40
    T = jnp.concatenate(
        [p, f.transpose(0, 2, 1),
         jnp.zeros((B, N, D - 3 - IN_C), jnp.float32)],
        axis=-1).reshape(B * N, D)
    X = _group_gather(T, gidx, new_p)
    x2 = X.reshape(B, M, NSAMPLE, D)
    W1p = jnp.pad(W1, ((0, 0), (0, D - W1.shape[1])))
    y = jnp.einsum('oc,bmkc->bomk', W1p, x2)
    mean = jnp.mean(y, axis=(0, 2, 3), keepdims=True)
    var = jnp.var(y, axis=(0, 2, 3), keepdims=True)
    y = (y - mean) / jnp.sqrt(var + 1e-5)
    y = jax.nn.relu(y * g1[None, :, None, None] + b1[None, :, None, None])
    x = _conv_bn_act(y, W2, g2, b2)
    f_out = _maxpool_k(x.reshape(B * OUT_C * M, NSAMPLE)).reshape(B, OUT_C, M)
    return (new_p, f_out)


# speedup vs baseline: 10.7139x; 1.1149x over previous
import functools
import jax
import jax.numpy as jnp
import numpy as np
from jax import lax
from jax.experimental import pallas as pl
from jax.experimental.pallas import tpu as pltpu
from jax.experimental.pallas import tpu_sc as plsc

IN_C = 32
OUT_C = 64
STRIDE = 4
RADIUS = 0.1
NSAMPLE = 32


def _fps_body(px_ref, py_ref, pz_ref, idx_ref, npx_ref, npy_ref, npz_ref,
              *, M, N, SUB, MSUB):
    LN = N // SUB
    ML = M // MSUB
    px = px_ref[0]
    py = py_ref[0]
    pz = pz_ref[0]
    nidx = (lax.broadcasted_iota(jnp.int32, (SUB, LN), 0) * LN
            + lax.broadcasted_iota(jnp.int32, (SUB, LN), 1))
    midx = (lax.broadcasted_iota(jnp.int32, (MSUB, ML), 0) * ML
            + lax.broadcasted_iota(jnp.int32, (MSUB, ML), 1))

    def body(m, st):
        dist, far, aidx, ax, ay, az = st
        sel = midx == m
        aidx = jnp.where(sel, far, aidx)
        onehot = nidx == far
        cx = jnp.sum(jnp.where(onehot, px, 0.0))
        cy = jnp.sum(jnp.where(onehot, py, 0.0))
        cz = jnp.sum(jnp.where(onehot, pz, 0.0))
        ax = jnp.where(sel, cx, ax)
        ay = jnp.where(sel, cy, ay)
        az = jnp.where(sel, cz, az)
        dxv = px - cx
        dyv = py - cy
        dzv = pz - cz
        d = (dxv * dxv + dzv * dzv) + dyv * dyv
        dist = jnp.minimum(dist, d)
        mx = jnp.max(dist)
        far = jnp.min(jnp.where(dist == mx, nidx, N))
        return dist, far, aidx, ax, ay, az

    dist0 = jnp.full((SUB, LN), 1e10, jnp.float32)
    z = jnp.zeros((MSUB, ML), jnp.float32)
    zi = jnp.zeros((MSUB, ML), jnp.int32)
    _, _, aidx, ax, ay, az = lax.fori_loop(
        0, M, body, (dist0, jnp.int32(0), zi, z, z, z))
    idx_ref[0] = aidx
    npx_ref[0] = ax
    npy_ref[0] = ay
    npz_ref[0] = az


def _fps_body2(px_ref, py_ref, pz_ref, idx_ref, npx_ref, npy_ref, npz_ref,
               *, M, N, B, SUB, MSUB):
    LN = N // SUB
    ML = M // MSUB
    pxs = [px_ref[pl.ds(b * SUB, SUB), :] for b in range(B)]
    pys = [py_ref[pl.ds(b * SUB, SUB), :] for b in range(B)]
    pzs = [pz_ref[pl.ds(b * SUB, SUB), :] for b in range(B)]
    nidx = (lax.broadcasted_iota(jnp.int32, (SUB, LN), 0) * LN
            + lax.broadcasted_iota(jnp.int32, (SUB, LN), 1))
    midx = (lax.broadcasted_iota(jnp.int32, (MSUB, ML), 0) * ML
            + lax.broadcasted_iota(jnp.int32, (MSUB, ML), 1))

    def body(m, st):
        dists, fars, aidxs, axs, ays, azs = st
        sel = midx == m
        out = ([], [], [], [], [], [])
        for b in range(B):
            dist, far = dists[b], fars[b]
            px, py, pz = pxs[b], pys[b], pzs[b]
            aidx = jnp.where(sel, far, aidxs[b])
            onehot = nidx == far
            cx = jnp.sum(jnp.where(onehot, px, 0.0))
            cy = jnp.sum(jnp.where(onehot, py, 0.0))
            cz = jnp.sum(jnp.where(onehot, pz, 0.0))
            ax = jnp.where(sel, cx, axs[b])
            ay = jnp.where(sel, cy, ays[b])
            az = jnp.where(sel, cz, azs[b])
            dxv = px - cx
            dyv = py - cy
            dzv = pz - cz
            d = (dxv * dxv + dzv * dzv) + dyv * dyv
            dist = jnp.minimum(dist, d)
            mx = jnp.max(dist)
            far = jnp.min(jnp.where(dist == mx, nidx, N))
            for lst, v in zip(out, (dist, far, aidx, ax, ay, az)):
                lst.append(v)
        return out

    dist0 = [jnp.full((SUB, LN), 1e10, jnp.float32)] * B
    far0 = [jnp.int32(0)] * B
    z = [jnp.zeros((MSUB, ML), jnp.float32)] * B
    zi = [jnp.zeros((MSUB, ML), jnp.int32)] * B
    _, _, aidxs, axs, ays, azs = lax.fori_loop(
        0, M, body, (dist0, far0, zi, z, z, z))
    for b in range(B):
        idx_ref[b] = aidxs[b]
        npx_ref[b] = axs[b]
        npy_ref[b] = ays[b]
        npz_ref[b] = azs[b]


def _fps(p, M):
    B, N, _ = p.shape
    SUB = 8
    MSUB = 16
    pr = p.transpose(0, 2, 1).reshape(B, 3, SUB, N // SUB)
    px = pr[:, 0].reshape(B * SUB, N // SUB)
    py = pr[:, 1].reshape(B * SUB, N // SUB)
    pz = pr[:, 2].reshape(B * SUB, N // SUB)
    idx, npx, npy, npz = pl.pallas_call(
        functools.partial(_fps_body2, M=M, N=N, B=B, SUB=SUB, MSUB=MSUB),
        out_shape=[
            jax.ShapeDtypeStruct((B, MSUB, M // MSUB), jnp.int32),
            jax.ShapeDtypeStruct((B, MSUB, M // MSUB), jnp.float32),
            jax.ShapeDtypeStruct((B, MSUB, M // MSUB), jnp.float32),
            jax.ShapeDtypeStruct((B, MSUB, M // MSUB), jnp.float32),
        ],
    )(px, py, pz)
    new_p = jnp.stack([npx.reshape(B, M), npy.reshape(B, M),
                       npz.reshape(B, M)], axis=-1)
    return idx.reshape(B, M), new_p


def _ball_query(p, new_p, radius, K):
    B, N, _ = p.shape
    M = new_p.shape[1]
    NW = 32
    CPT = (B * M) // NW
    PW = NW // B
    CH = 128
    NCH = N // CH
    CBUF = 128
    r2 = np.float32(radius * radius)

    mesh = plsc.VectorSubcoreMesh(core_axis_name="c", subcore_axis_name="s")

    @functools.partial(
        pl.kernel,
        out_type=jax.ShapeDtypeStruct((B * M * K,), jnp.int32),
        mesh=mesh,
        scratch_types=[
            pltpu.VMEM((N,), jnp.float32),
            pltpu.VMEM((N,), jnp.float32),
            pltpu.VMEM((N,), jnp.float32),
            pltpu.VMEM((CPT + 16,), jnp.float32),
            pltpu.VMEM((CPT + 16,), jnp.float32),
            pltpu.VMEM((CPT + 16,), jnp.float32),
            pltpu.VMEM((CBUF,), jnp.int32),
            pltpu.VMEM((CPT * K,), jnp.int32),
        ],
    )
    def bq(px_h, py_h, pz_h, cx_h, cy_h, cz_h, out_h,
           pxv, pyv, pzv, cxv, cyv, czv, cand, gbuf):
        wid = lax.axis_index("s") * 2 + lax.axis_index("c")
        b = wid // PW
        part = wid % PW
        pltpu.sync_copy(px_h.at[pl.ds(b * N, N)], pxv)
        pltpu.sync_copy(py_h.at[pl.ds(b * N, N)], pyv)
        pltpu.sync_copy(pz_h.at[pl.ds(b * N, N)], pzv)
        cbase = b * M + part * CPT
        pltpu.sync_copy(cx_h.at[pl.ds(cbase, CPT)], cxv.at[pl.ds(0, CPT)])
        pltpu.sync_copy(cy_h.at[pl.ds(cbase, CPT)], cyv.at[pl.ds(0, CPT)])
        pltpu.sync_copy(cz_h.at[pl.ds(cbase, CPT)], czv.at[pl.ds(0, CPT)])

        lanes = lax.iota(jnp.int32, 16)
        zeros16 = jnp.zeros((16,), jnp.int32)

        bofs = b * N

        def per_centroid(i, carry):
            ccx = cxv[pl.ds(i, 16)][0]
            ccy = cyv[pl.ds(i, 16)][0]
            ccz = czv[pl.ds(i, 16)][0]
            cand[pl.ds(0, 16)] = zeros16 + bofs

            one16 = jnp.ones((16,), jnp.int32)
            zero16 = jnp.zeros((16,), jnp.int32)

            def hsum(v):
                s = v[0]
                for l in range(1, 16):
                    s = s + v[l]
                return s

            BIG = jnp.int32(1 << 30)
            big16 = jnp.zeros((16,), jnp.int32) + BIG

            def chunk(j, cnt):
                base = j * CH
                mi = []
                for r in range(CH // 16):
                    off = base + r * 16
                    dx = pxv[pl.ds(off, 16)] - ccx
                    dy = pyv[pl.ds(off, 16)] - ccy
                    dz = pzv[pl.ds(off, 16)] - ccz
                    d2 = (dx * dx + dz * dz) + dy * dy
                    mi.append(jnp.where(d2 <= r2, one16, zero16))
                tot = mi[0]
                for r in range(1, CH // 16):
                    tot = tot + mi[r]
                s1 = hsum(tot)

                def one_hit(cnt):
                    wv = mi[0] * (lanes + base)
                    for r in range(1, CH // 16):
                        wv = wv + mi[r] * (lanes + (base + r * 16))
                    gidx = hsum(wv) + bofs
                    app = jnp.where(lanes == 0, gidx, BIG)
                    cand[pl.ds(jnp.minimum(cnt, 64), 16)] = app
                    return cnt + 1

                def multi_hit(cnt):
                    for r in range(CH // 16):
                        v = big16
                        rank = jnp.int32(0)
                        for l in range(16):
                            ml = mi[r][l]
                            val = jnp.where(ml > 0,
                                            bofs + (base + r * 16 + l),
                                            BIG)
                            v = jnp.where(lanes == rank, val, v)
                            rank = rank + ml
                        cand[pl.ds(jnp.minimum(cnt, 64), 16)] = v
                        cnt = cnt + rank
                    return cnt

                def any_hit(cnt):
                    return lax.cond(s1 == 1, one_hit, multi_hit, cnt)

                return lax.cond(s1 > 0, any_hit, lambda c: c, cnt)

            cnt = lax.fori_loop(0, NCH, chunk, jnp.int32(0))

            first = jnp.zeros((16,), jnp.int32) + cand[pl.ds(0, 16)][0]
            row0 = jnp.where(lanes < cnt, cand[pl.ds(0, 16)], first)
            row1 = jnp.where(lanes + 16 < cnt, cand[pl.ds(16, 16)], first)
            gbuf[pl.ds(i * K, 16)] = row0
            gbuf[pl.ds(i * K + 16, 16)] = row1
            return carry

        lax.fori_loop(0, CPT, per_centroid, jnp.int32(0))
        pltpu.sync_copy(gbuf, out_h.at[pl.ds(wid * CPT * K, CPT * K)])

    pT = p.transpose(2, 0, 1).reshape(3, B * N)
    cT = new_p.transpose(2, 0, 1).reshape(3, B * M)
    return bq(pT[0], pT[1], pT[2], cT[0], cT[1], cT[2])


def _group_gather(T, gidx_flat, new_p):
    BN, D = T.shape
    B, M, _ = new_p.shape
    R = gidx_flat.shape[0]
    K = R // (B * M)
    NW = 32
    CPT = (B * M) // NW
    RPT = CPT * K
    SUB = 32
    SROWS = SUB * K

    mesh = plsc.VectorSubcoreMesh(core_axis_name="c", subcore_axis_name="s")

    @functools.partial(
        pl.kernel,
        out_type=jax.ShapeDtypeStruct((R, D), jnp.float32),
        mesh=mesh,
        scratch_types=[
            pltpu.VMEM((RPT,), jnp.int32),
            pltpu.VMEM((CPT + 16,), jnp.float32),
            pltpu.VMEM((CPT + 16,), jnp.float32),
            pltpu.VMEM((CPT + 16,), jnp.float32),
            pltpu.VMEM((SROWS, D), jnp.float32),
            pltpu.SemaphoreType.DMA,
        ],
        compiler_params=pltpu.CompilerParams(use_tc_tiling_on_sc=False),
    )
    def gg(t_h, gi_h, cx_h, cy_h, cz_h, x_h, giv, cxv, cyv, czv, xbuf, sem):
        wid = lax.axis_index("s") * 2 + lax.axis_index("c")
        pltpu.sync_copy(gi_h.at[pl.ds(wid * RPT, RPT)], giv)
        cb = wid * CPT
        pltpu.sync_copy(cx_h.at[pl.ds(cb, CPT)], cxv.at[pl.ds(0, CPT)])
        pltpu.sync_copy(cy_h.at[pl.ds(cb, CPT)], cyv.at[pl.ds(0, CPT)])
        pltpu.sync_copy(cz_h.at[pl.ds(cb, CPT)], czv.at[pl.ds(0, CPT)])

        lanes = lax.iota(jnp.int32, 16)
        zf = jnp.zeros((16,), jnp.float32)

        def sub_chunk(s, carry):
            cps = []
            for q in range(SROWS // 128):
                idxs = giv.at[pl.ds(s * SROWS + q * 128, 128)]
                cps.append(pltpu.async_copy(
                    t_h.at[idxs], xbuf.at[pl.ds(q * 128, 128)], sem))
            for cp in cps:
                cp.wait()
            def fix_c(ci, carry2):
                c = s * SUB + ci
                ccx = cxv[pl.ds(c, 16)][0]
                ccy = cyv[pl.ds(c, 16)][0]
                ccz = czv[pl.ds(c, 16)][0]
                cvec = jnp.where(lanes == 0, ccx,
                                 jnp.where(lanes == 1, ccy,
                                           jnp.where(lanes == 2, ccz, zf)))
                for r in range(K):
                    row = ci * K + r
                    v = xbuf[row, pl.ds(0, 16)]
                    xbuf[row, pl.ds(0, 16)] = v - cvec
                return carry2

            lax.fori_loop(0, SUB, fix_c, jnp.int32(0))
            pltpu.sync_copy(
                xbuf, x_h.at[pl.ds(wid * RPT + s * SROWS, SROWS)])
            return carry

        lax.fori_loop(0, CPT // SUB, sub_chunk, jnp.int32(0))

    cT = new_p.transpose(2, 0, 1).reshape(3, B * M)
    return gg(T, gidx_flat, cT[0], cT[1], cT[2])


def _conv_bn_act(x, W, gamma, beta):
    y = jnp.einsum('oc,bcmk->bomk', W, x)
    mean = jnp.mean(y, axis=(0, 2, 3), keepdims=True)
    var = jnp.var(y, axis=(0, 2, 3), keepdims=True)
    y = (y - mean) / jnp.sqrt(var + 1e-5)
    y = y * gamma[None, :, None, None] + beta[None, :, None, None]
    return jax.nn.relu(y)


def _maxpool_body(x_ref, o_ref):
    o_ref[...] = jnp.max(x_ref[...], axis=1)[:, None]


def _maxpool_k(x):
    R, K = x.shape
    blk = 1024
    out = pl.pallas_call(
        _maxpool_body,
        grid=(R // blk,),
        in_specs=[pl.BlockSpec((blk, K), lambda i: (i, 0))],
        out_specs=pl.BlockSpec((blk, 1), lambda i: (i, 0)),
        out_shape=jax.ShapeDtypeStruct((R, 1), x.dtype),
    )(x)
    return out[:, 0]


def kernel(p, f, W1, g1, b1, W2, g2, b2):
    B, N, _ = p.shape
    M = N // STRIDE
    idx, new_p = _fps(p, M)
    del idx
    gidx = _ball_query(p, new_p, RADIUS, NSAMPLE)
    D = 40
    T = jnp.concatenate(
        [p, f.transpose(0, 2, 1),
         jnp.zeros((B, N, D - 3 - IN_C), jnp.float32)],
        axis=-1).reshape(B * N, D)
    X = _group_gather(T, gidx, new_p)
    x2 = X.reshape(B, M, NSAMPLE, D)
    W1p = jnp.pad(W1, ((0, 0), (0, D - W1.shape[1])))
    y = jnp.einsum('oc,bmkc->bomk', W1p, x2)
    mean = jnp.mean(y, axis=(0, 2, 3), keepdims=True)
    var = jnp.var(y, axis=(0, 2, 3), keepdims=True)
    y = (y - mean) / jnp.sqrt(var + 1e-5)
    y = jax.nn.relu(y * g1[None, :, None, None] + b1[None, :, None, None])
    x = _conv_bn_act(y, W2, g2, b2)
    f_out = jnp.max(x, axis=-1)
    return (new_p, f_out)


# speedup vs baseline: 10.7163x; 1.0002x over previous
import functools
import jax
import jax.numpy as jnp
import numpy as np
from jax import lax
from jax.experimental import pallas as pl
from jax.experimental.pallas import tpu as pltpu
from jax.experimental.pallas import tpu_sc as plsc

IN_C = 32
OUT_C = 64
STRIDE = 4
RADIUS = 0.1
NSAMPLE = 32


def _fps_body2(px_ref, py_ref, pz_ref, idx_ref, npx_ref, npy_ref, npz_ref,
               *, M, N, B, SUB, MSUB):
    LN = N // SUB
    ML = M // MSUB
    pxs = [px_ref[pl.ds(b * SUB, SUB), :] for b in range(B)]
    pys = [py_ref[pl.ds(b * SUB, SUB), :] for b in range(B)]
    pzs = [pz_ref[pl.ds(b * SUB, SUB), :] for b in range(B)]
    nidx = (lax.broadcasted_iota(jnp.int32, (SUB, LN), 0) * LN
            + lax.broadcasted_iota(jnp.int32, (SUB, LN), 1))
    midx = (lax.broadcasted_iota(jnp.int32, (MSUB, ML), 0) * ML
            + lax.broadcasted_iota(jnp.int32, (MSUB, ML), 1))

    def body(m, st):
        dists, fars, aidxs, axs, ays, azs = st
        sel = midx == m
        out = ([], [], [], [], [], [])
        for b in range(B):
            dist, far = dists[b], fars[b]
            px, py, pz = pxs[b], pys[b], pzs[b]
            aidx = jnp.where(sel, far, aidxs[b])
            onehot = nidx == far
            cx = jnp.sum(jnp.where(onehot, px, 0.0))
            cy = jnp.sum(jnp.where(onehot, py, 0.0))
            cz = jnp.sum(jnp.where(onehot, pz, 0.0))
            ax = jnp.where(sel, cx, axs[b])
            ay = jnp.where(sel, cy, ays[b])
            az = jnp.where(sel, cz, azs[b])
            dxv = px - cx
            dyv = py - cy
            dzv = pz - cz
            d = (dxv * dxv + dzv * dzv) + dyv * dyv
            dist = jnp.minimum(dist, d)
            mx = jnp.max(dist)
            far = jnp.min(jnp.where(dist == mx, nidx, N))
            for lst, v in zip(out, (dist, far, aidx, ax, ay, az)):
                lst.append(v)
        return out

    dist0 = [jnp.full((SUB, LN), 1e10, jnp.float32)] * B
    far0 = [jnp.int32(0)] * B
    z = [jnp.zeros((MSUB, ML), jnp.float32)] * B
    zi = [jnp.zeros((MSUB, ML), jnp.int32)] * B
    _, _, aidxs, axs, ays, azs = lax.fori_loop(
        0, M, body, (dist0, far0, zi, z, z, z))
    for b in range(B):
        idx_ref[b] = aidxs[b]
        npx_ref[b] = axs[b]
        npy_ref[b] = ays[b]
        npz_ref[b] = azs[b]


def _fps(p, M):
    B, N, _ = p.shape
    SUB = 8
    MSUB = 16
    pr = p.transpose(0, 2, 1).reshape(B, 3, SUB, N // SUB)
    px = pr[:, 0].reshape(B * SUB, N // SUB)
    py = pr[:, 1].reshape(B * SUB, N // SUB)
    pz = pr[:, 2].reshape(B * SUB, N // SUB)
    idx, npx, npy, npz = pl.pallas_call(
        functools.partial(_fps_body2, M=M, N=N, B=B, SUB=SUB, MSUB=MSUB),
        out_shape=[
            jax.ShapeDtypeStruct((B, MSUB, M // MSUB), jnp.int32),
            jax.ShapeDtypeStruct((B, MSUB, M // MSUB), jnp.float32),
            jax.ShapeDtypeStruct((B, MSUB, M // MSUB), jnp.float32),
            jax.ShapeDtypeStruct((B, MSUB, M // MSUB), jnp.float32),
        ],
    )(px, py, pz)
    new_p = jnp.stack([npx.reshape(B, M), npy.reshape(B, M),
                       npz.reshape(B, M)], axis=-1)
    return idx.reshape(B, M), new_p


def _ball_query(p, new_p, radius, K):
    B, N, _ = p.shape
    M = new_p.shape[1]
    NW = 32
    CPT = (B * M) // NW
    PW = NW // B
    CH = 128
    NCH = N // CH
    CBUF = 128
    r2 = np.float32(radius * radius)

    mesh = plsc.VectorSubcoreMesh(core_axis_name="c", subcore_axis_name="s")

    @functools.partial(
        pl.kernel,
        out_type=jax.ShapeDtypeStruct((B * M * K,), jnp.int32),
        mesh=mesh,
        scratch_types=[
            pltpu.VMEM((N,), jnp.float32),
            pltpu.VMEM((N,), jnp.float32),
            pltpu.VMEM((N,), jnp.float32),
            pltpu.VMEM((CPT + 16,), jnp.float32),
            pltpu.VMEM((CPT + 16,), jnp.float32),
            pltpu.VMEM((CPT + 16,), jnp.float32),
            pltpu.VMEM((CBUF,), jnp.int32),
            pltpu.VMEM((CPT * K,), jnp.int32),
        ],
    )
    def bq(px_h, py_h, pz_h, cx_h, cy_h, cz_h, out_h,
           pxv, pyv, pzv, cxv, cyv, czv, cand, gbuf):
        wid = lax.axis_index("s") * 2 + lax.axis_index("c")
        b = wid // PW
        part = wid % PW
        pltpu.sync_copy(px_h.at[pl.ds(b * N, N)], pxv)
        pltpu.sync_copy(py_h.at[pl.ds(b * N, N)], pyv)
        pltpu.sync_copy(pz_h.at[pl.ds(b * N, N)], pzv)
        cbase = b * M + part * CPT
        pltpu.sync_copy(cx_h.at[pl.ds(cbase, CPT)], cxv.at[pl.ds(0, CPT)])
        pltpu.sync_copy(cy_h.at[pl.ds(cbase, CPT)], cyv.at[pl.ds(0, CPT)])
        pltpu.sync_copy(cz_h.at[pl.ds(cbase, CPT)], czv.at[pl.ds(0, CPT)])

        lanes = lax.iota(jnp.int32, 16)
        zeros16 = jnp.zeros((16,), jnp.int32)

        bofs = b * N

        def per_centroid(i, carry):
            ccx = cxv[pl.ds(i, 16)][0]
            ccy = cyv[pl.ds(i, 16)][0]
            ccz = czv[pl.ds(i, 16)][0]
            cand[pl.ds(0, 16)] = zeros16 + bofs

            one16 = jnp.ones((16,), jnp.int32)
            zero16 = jnp.zeros((16,), jnp.int32)

            def hsum(v):
                s = v[0]
                for l in range(1, 16):
                    s = s + v[l]
                return s

            BIG = jnp.int32(1 << 30)
            big16 = jnp.zeros((16,), jnp.int32) + BIG

            def chunk(j, cnt):
                base = j * CH
                mi = []
                for r in range(CH // 16):
                    off = base + r * 16
                    dx = pxv[pl.ds(off, 16)] - ccx
                    dy = pyv[pl.ds(off, 16)] - ccy
                    dz = pzv[pl.ds(off, 16)] - ccz
                    d2 = (dx * dx + dz * dz) + dy * dy
                    mi.append(jnp.where(d2 <= r2, one16, zero16))
                tot = mi[0]
                for r in range(1, CH // 16):
                    tot = tot + mi[r]
                s1 = hsum(tot)

                def one_hit(cnt):
                    wv = mi[0] * (lanes + base)
                    for r in range(1, CH // 16):
                        wv = wv + mi[r] * (lanes + (base + r * 16))
                    gidx = hsum(wv) + bofs
                    app = jnp.where(lanes == 0, gidx, BIG)
                    cand[pl.ds(jnp.minimum(cnt, 64), 16)] = app
                    return cnt + 1

                def multi_hit(cnt):
                    for r in range(CH // 16):
                        v = big16
                        rank = jnp.int32(0)
                        for l in range(16):
                            ml = mi[r][l]
                            val = jnp.where(ml > 0,
                                            bofs + (base + r * 16 + l),
                                            BIG)
                            v = jnp.where(lanes == rank, val, v)
                            rank = rank + ml
                        cand[pl.ds(jnp.minimum(cnt, 64), 16)] = v
                        cnt = cnt + rank
                    return cnt

                def any_hit(cnt):
                    return lax.cond(s1 == 1, one_hit, multi_hit, cnt)

                return lax.cond(s1 > 0, any_hit, lambda c: c, cnt)

            cnt = lax.fori_loop(0, NCH, chunk, jnp.int32(0))

            first = jnp.zeros((16,), jnp.int32) + cand[pl.ds(0, 16)][0]
            row0 = jnp.where(lanes < cnt, cand[pl.ds(0, 16)], first)
            row1 = jnp.where(lanes + 16 < cnt, cand[pl.ds(16, 16)], first)
            gbuf[pl.ds(i * K, 16)] = row0
            gbuf[pl.ds(i * K + 16, 16)] = row1
            return carry

        lax.fori_loop(0, CPT, per_centroid, jnp.int32(0))
        pltpu.sync_copy(gbuf, out_h.at[pl.ds(wid * CPT * K, CPT * K)])

    pT = p.transpose(2, 0, 1).reshape(3, B * N)
    cT = new_p.transpose(2, 0, 1).reshape(3, B * M)
    return bq(pT[0], pT[1], pT[2], cT[0], cT[1], cT[2])


def _group_gather(T, gidx_flat, new_p):
    BN, D = T.shape
    B, M, _ = new_p.shape
    R = gidx_flat.shape[0]
    K = R // (B * M)
    NW = 32
    CPT = (B * M) // NW
    RPT = CPT * K
    SUB = 32
    SROWS = SUB * K

    mesh = plsc.VectorSubcoreMesh(core_axis_name="c", subcore_axis_name="s")

    @functools.partial(
        pl.kernel,
        out_type=jax.ShapeDtypeStruct((R, D), jnp.float32),
        mesh=mesh,
        scratch_types=[
            pltpu.VMEM((RPT,), jnp.int32),
            pltpu.VMEM((CPT + 16,), jnp.float32),
            pltpu.VMEM((CPT + 16,), jnp.float32),
            pltpu.VMEM((CPT + 16,), jnp.float32),
            pltpu.VMEM((SROWS, D), jnp.float32),
            pltpu.SemaphoreType.DMA,
        ],
        compiler_params=pltpu.CompilerParams(use_tc_tiling_on_sc=False),
    )
    def gg(t_h, gi_h, cx_h, cy_h, cz_h, x_h, giv, cxv, cyv, czv, xbuf, sem):
        wid = lax.axis_index("s") * 2 + lax.axis_index("c")
        pltpu.sync_copy(gi_h.at[pl.ds(wid * RPT, RPT)], giv)
        cb = wid * CPT
        pltpu.sync_copy(cx_h.at[pl.ds(cb, CPT)], cxv.at[pl.ds(0, CPT)])
        pltpu.sync_copy(cy_h.at[pl.ds(cb, CPT)], cyv.at[pl.ds(0, CPT)])
        pltpu.sync_copy(cz_h.at[pl.ds(cb, CPT)], czv.at[pl.ds(0, CPT)])

        lanes = lax.iota(jnp.int32, 16)
        zf = jnp.zeros((16,), jnp.float32)

        def sub_chunk(s, carry):
            cps = []
            for q in range(SROWS // 128):
                idxs = giv.at[pl.ds(s * SROWS + q * 128, 128)]
                cps.append(pltpu.async_copy(
                    t_h.at[idxs], xbuf.at[pl.ds(q * 128, 128)], sem))
            for cp in cps:
                cp.wait()
            def fix_c(ci, carry2):
                c = s * SUB + ci
                ccx = cxv[pl.ds(c, 16)][0]
                ccy = cyv[pl.ds(c, 16)][0]
                ccz = czv[pl.ds(c, 16)][0]
                cvec = jnp.where(lanes == 0, ccx,
                                 jnp.where(lanes == 1, ccy,
                                           jnp.where(lanes == 2, ccz, zf)))
                for r in range(K):
                    row = ci * K + r
                    v = xbuf[row, pl.ds(0, 16)]
                    xbuf[row, pl.ds(0, 16)] = v - cvec
                return carry2

            lax.fori_loop(0, SUB, fix_c, jnp.int32(0))
            pltpu.sync_copy(
                xbuf, x_h.at[pl.ds(wid * RPT + s * SROWS, SROWS)])
            return carry

        lax.fori_loop(0, CPT // SUB, sub_chunk, jnp.int32(0))

    cT = new_p.transpose(2, 0, 1).reshape(3, B * M)
    return gg(T, gidx_flat, cT[0], cT[1], cT[2])


def _conv_bn_act(x, W, gamma, beta):
    y = jnp.einsum('oc,bcmk->bomk', W, x)
    mean = jnp.mean(y, axis=(0, 2, 3), keepdims=True)
    var = jnp.var(y, axis=(0, 2, 3), keepdims=True)
    y = (y - mean) / jnp.sqrt(var + 1e-5)
    y = y * gamma[None, :, None, None] + beta[None, :, None, None]
    return jax.nn.relu(y)


def kernel(p, f, W1, g1, b1, W2, g2, b2):
    B, N, _ = p.shape
    M = N // STRIDE
    idx, new_p = _fps(p, M)
    del idx
    gidx = _ball_query(p, new_p, RADIUS, NSAMPLE)
    D = 40
    T = jnp.concatenate(
        [p, f.transpose(0, 2, 1),
         jnp.zeros((B, N, D - 3 - IN_C), jnp.float32)],
        axis=-1).reshape(B * N, D)
    X = _group_gather(T, gidx, new_p)
    x2 = X.reshape(B, M, NSAMPLE, D)
    W1p = jnp.pad(W1, ((0, 0), (0, D - W1.shape[1])))
    y = jnp.einsum('oc,bmkc->bomk', W1p, x2)
    mean = jnp.mean(y, axis=(0, 2, 3), keepdims=True)
    var = jnp.var(y, axis=(0, 2, 3), keepdims=True)
    y = (y - mean) / jnp.sqrt(var + 1e-5)
    y = jax.nn.relu(y * g1[None, :, None, None] + b1[None, :, None, None])
    x = _conv_bn_act(y, W2, g2, b2)
    f_out = jnp.max(x, axis=-1)
    return (new_p, f_out)
